# Initial kernel scaffold; baseline (speedup 1.0000x reference)
import jax
import jax.numpy as jnp
from jax.experimental import pallas as pl


def kernel(x_user, x_item, edge_index_user_to_item, edge_index_item_to_user, W_l_u2i, b_l_u2i, W_r_u2i, W_l_i2u, b_l_i2u, W_r_i2u):
    raise NotImplementedError("write your pallas kernel here")


# speedup vs baseline: 4.5921x; 4.5921x over previous
import functools

import jax
import jax.numpy as jnp
from jax import lax
from jax.experimental import pallas as pl
from jax.experimental.pallas import tpu as pltpu
from jax.experimental.pallas import tpu_sc as plsc

N_NODE = 10000
D = 128
E_EDGES = 320000

NS = 16
N_PAD = 10240
ROWS_PER_TILE = N_PAD // NS
CHUNK = 80
EDGES_PER_TILE = E_EDGES // NS
NCHUNK = EDGES_PER_TILE // CHUNK
ZROWS = 128


@functools.partial(
    pl.kernel,
    mesh=plsc.VectorSubcoreMesh(core_axis_name="c", subcore_axis_name="s"),
    out_type=jax.ShapeDtypeStruct((2 * N_PAD, D), jnp.float32),
    scratch_types=[
        pltpu.VMEM((CHUNK,), jnp.int32),
        pltpu.VMEM((CHUNK,), jnp.int32),
        pltpu.VMEM((CHUNK, D), jnp.float32),
        pltpu.VMEM((ZROWS, D), jnp.float32),
        pltpu.VMEM_SHARED((N_PAD, D), jnp.float32),
        pltpu.SemaphoreType.DMA,
    ],
)
def _sc_agg(x_hbm, src_hbm, dst_hbm, zeros_hbm, agg_hbm,
            sidx, didx, rows, zbuf, aggsh, sem):
    c = lax.axis_index("c")
    s = lax.axis_index("s")

    rowbase = s * ROWS_PER_TILE
    pltpu.sync_copy(zeros_hbm, zbuf)

    def _zero(k, carry):
        pltpu.sync_copy(zbuf, aggsh.at[pl.ds(rowbase + k * ZROWS, ZROWS)])
        return carry

    lax.fori_loop(0, ROWS_PER_TILE // ZROWS, _zero, 0)
    plsc.subcore_barrier()

    ebase = c * E_EDGES + s * EDGES_PER_TILE

    def _edges(g, carry):
        off = ebase + g * CHUNK
        pltpu.sync_copy(src_hbm.at[pl.ds(off, CHUNK)], sidx)
        pltpu.sync_copy(dst_hbm.at[pl.ds(off, CHUNK)], didx)
        pltpu.async_copy(x_hbm.at[sidx], rows, sem).wait()
        pltpu.sync_copy(rows, aggsh.at[didx], add=True)
        return carry

    lax.fori_loop(0, NCHUNK, _edges, 0)
    plsc.subcore_barrier()

    outbase = c * N_PAD + rowbase

    def _readback(k, carry):
        pltpu.sync_copy(aggsh.at[pl.ds(rowbase + k * ZROWS, ZROWS)], zbuf)
        pltpu.sync_copy(zbuf, agg_hbm.at[pl.ds(outbase + k * ZROWS, ZROWS)])
        return carry

    lax.fori_loop(0, ROWS_PER_TILE // ZROWS, _readback, 0)


BLK = 1000


def _dense_body(agg_u, x_u, agg_i, x_i, wl_u, b_u, wr_u, wl_i, b_i, wr_i,
                out_u, out_i):
    f32 = jnp.float32
    out_u[...] = (jnp.dot(agg_u[...], wl_u[...], preferred_element_type=f32)
                  + jnp.dot(x_u[...], wr_u[...], preferred_element_type=f32)
                  + b_u[...])
    out_i[...] = (jnp.dot(agg_i[...], wl_i[...], preferred_element_type=f32)
                  + jnp.dot(x_i[...], wr_i[...], preferred_element_type=f32)
                  + b_i[...])


def _dense(agg_user, x_user, agg_item, x_item,
           WlT_u, b_u, WrT_u, WlT_i, b_i, WrT_i):
    row_spec = pl.BlockSpec((BLK, D), lambda i: (i, 0))
    w_spec = pl.BlockSpec((D, D), lambda i: (0, 0))
    b_spec = pl.BlockSpec((1, D), lambda i: (0, 0))
    return pl.pallas_call(
        _dense_body,
        grid=(N_NODE // BLK,),
        in_specs=[row_spec, row_spec, row_spec, row_spec,
                  w_spec, b_spec, w_spec, w_spec, b_spec, w_spec],
        out_specs=[row_spec, row_spec],
        out_shape=[jax.ShapeDtypeStruct((N_NODE, D), jnp.float32),
                   jax.ShapeDtypeStruct((N_NODE, D), jnp.float32)],
    )(agg_user, x_user, agg_item, x_item,
      WlT_u, b_u, WrT_u, WlT_i, b_i, WrT_i)


def kernel(x_user, x_item, edge_index_user_to_item, edge_index_item_to_user,
           W_l_u2i, b_l_u2i, W_r_u2i, W_l_i2u, b_l_i2u, W_r_i2u):
    x_all = jnp.concatenate([x_user, x_item], axis=0)
    src_all = jnp.concatenate([edge_index_user_to_item[0],
                               edge_index_item_to_user[0] + N_NODE])
    dst_all = jnp.concatenate([edge_index_user_to_item[1],
                               edge_index_item_to_user[1]])
    zeros_rows = jnp.zeros((ZROWS, D), jnp.float32)

    agg_flat = _sc_agg(x_all, src_all, dst_all, zeros_rows)
    agg_item = agg_flat[0:N_NODE]
    agg_user = agg_flat[N_PAD:N_PAD + N_NODE]

    out_user, out_item = _dense(
        agg_user, x_user, agg_item, x_item,
        W_l_i2u.T, b_l_i2u.reshape(1, D), W_r_i2u.T,
        W_l_u2i.T, b_l_u2i.reshape(1, D), W_r_u2i.T)
    return out_user, out_item


# speedup vs baseline: 8.3259x; 1.8131x over previous
import functools

import jax
import jax.numpy as jnp
from jax import lax
from jax.experimental import pallas as pl
from jax.experimental.pallas import tpu as pltpu
from jax.experimental.pallas import tpu_sc as plsc

N_NODE = 10000
D = 128
E_EDGES = 320000

NS = 16
N_PAD = 10240
ROWS_PER_TILE = N_PAD // NS
CHUNK = 125
ROWS_PER_REL = E_EDGES // CHUNK
CHUNKS_PER_TILE = ROWS_PER_REL // NS
NI = 16
NBLK = CHUNKS_PER_TILE // NI
NITER = CHUNKS_PER_TILE // 2


@functools.partial(
    pl.kernel,
    mesh=plsc.VectorSubcoreMesh(core_axis_name="c", subcore_axis_name="s"),
    out_type=jax.ShapeDtypeStruct((2 * N_PAD, D), jnp.float32),
    scratch_types=[
        pltpu.VMEM((2, NI, CHUNK), jnp.int32),
        pltpu.VMEM((2, NI, CHUNK), jnp.int32),
        pltpu.VMEM((CHUNK, D), jnp.float32),
        pltpu.VMEM((CHUNK, D), jnp.float32),
        pltpu.VMEM_SHARED((N_PAD, D), jnp.float32),
        pltpu.SemaphoreType.DMA,
        pltpu.SemaphoreType.DMA,
        pltpu.SemaphoreType.DMA,
        pltpu.SemaphoreType.DMA,
        pltpu.SemaphoreType.DMA,
    ],
)
def _sc_agg(x_hbm, src_hbm, dst_hbm, zeros_hbm, agg_hbm,
            sidx, didx, r0, r1, aggsh, g0, g1, s0, s1, isem):
    c = lax.axis_index("c")
    s = lax.axis_index("s")

    rowbase = s * ROWS_PER_TILE
    pltpu.sync_copy(zeros_hbm, aggsh.at[pl.ds(rowbase, ROWS_PER_TILE)])
    plsc.subcore_barrier()

    tbase = c * ROWS_PER_REL + s * CHUNKS_PER_TILE

    def _load_block(b, slot, sem):
        pltpu.async_copy(src_hbm.at[pl.ds(tbase + NI * b, NI)], sidx.at[slot],
                         sem)
        pltpu.async_copy(dst_hbm.at[pl.ds(tbase + NI * b, NI)], didx.at[slot],
                         sem)

    def _wait_block(slot, sem):
        pltpu.make_async_copy(src_hbm.at[pl.ds(tbase, NI)], sidx.at[slot],
                              sem).wait()
        pltpu.make_async_copy(dst_hbm.at[pl.ds(tbase, NI)], didx.at[slot],
                              sem).wait()

    _load_block(0, 0, isem)
    _wait_block(0, isem)
    _load_block(1, 1, isem)

    pltpu.async_copy(x_hbm.at[sidx.at[0, 0]], r0, g0)
    pltpu.async_copy(x_hbm.at[sidx.at[0, 1]], r1, g1)

    def _steady(jj, carry):
        slot = (jj // 8) % 2
        row0 = 2 * (jj % 8)
        pltpu.make_async_copy(x_hbm.at[sidx.at[slot, row0]], r0, g0).wait()
        pltpu.async_copy(r0, aggsh.at[didx.at[slot, row0]], s0, add=True)
        pltpu.make_async_copy(x_hbm.at[sidx.at[slot, row0 + 1]], r1, g1).wait()
        pltpu.async_copy(r1, aggsh.at[didx.at[slot, row0 + 1]], s1, add=True)

        @pl.when(jj < NITER - 1)
        def _():
            pltpu.make_async_copy(r0, aggsh.at[didx.at[slot, row0]], s0).wait()
            pltpu.make_async_copy(r1, aggsh.at[didx.at[slot, row0 + 1]],
                                  s1).wait()

            @pl.when(jnp.logical_and(jj % 8 == 7, jj < 8 * (NBLK - 2)))
            def _():
                _load_block(jj // 8 + 2, slot, isem)

            @pl.when(jj % 8 == 7)
            def _():
                _wait_block(1 - slot, isem)

            nslot = ((jj + 1) // 8) % 2
            nrow0 = 2 * ((jj + 1) % 8)
            pltpu.async_copy(x_hbm.at[sidx.at[nslot, nrow0]], r0, g0)
            pltpu.async_copy(x_hbm.at[sidx.at[nslot, nrow0 + 1]], r1, g1)

        return carry

    lax.fori_loop(0, NITER, _steady, 0)
    pltpu.make_async_copy(r0, aggsh.at[didx.at[1, NI - 2]], s0).wait()
    pltpu.make_async_copy(r1, aggsh.at[didx.at[1, NI - 1]], s1).wait()
    plsc.subcore_barrier()

    outbase = c * N_PAD + rowbase
    pltpu.sync_copy(aggsh.at[pl.ds(rowbase, ROWS_PER_TILE)],
                    agg_hbm.at[pl.ds(outbase, ROWS_PER_TILE)])


BLK = 1000


def _dense_body(agg_u, x_u, agg_i, x_i, wl_u, b_u, wr_u, wl_i, b_i, wr_i,
                out_u, out_i):
    f32 = jnp.float32
    out_u[...] = (jnp.dot(agg_u[...], wl_u[...], preferred_element_type=f32)
                  + jnp.dot(x_u[...], wr_u[...], preferred_element_type=f32)
                  + b_u[...])
    out_i[...] = (jnp.dot(agg_i[...], wl_i[...], preferred_element_type=f32)
                  + jnp.dot(x_i[...], wr_i[...], preferred_element_type=f32)
                  + b_i[...])


def _dense(agg_user, x_user, agg_item, x_item,
           WlT_u, b_u, WrT_u, WlT_i, b_i, WrT_i):
    row_spec = pl.BlockSpec((BLK, D), lambda i: (i, 0))
    w_spec = pl.BlockSpec((D, D), lambda i: (0, 0))
    b_spec = pl.BlockSpec((1, D), lambda i: (0, 0))
    return pl.pallas_call(
        _dense_body,
        grid=(N_NODE // BLK,),
        in_specs=[row_spec, row_spec, row_spec, row_spec,
                  w_spec, b_spec, w_spec, w_spec, b_spec, w_spec],
        out_specs=[row_spec, row_spec],
        out_shape=[jax.ShapeDtypeStruct((N_NODE, D), jnp.float32),
                   jax.ShapeDtypeStruct((N_NODE, D), jnp.float32)],
    )(agg_user, x_user, agg_item, x_item,
      WlT_u, b_u, WrT_u, WlT_i, b_i, WrT_i)


def kernel(x_user, x_item, edge_index_user_to_item, edge_index_item_to_user,
           W_l_u2i, b_l_u2i, W_r_u2i, W_l_i2u, b_l_i2u, W_r_i2u):
    x_all = jnp.concatenate([x_user, x_item], axis=0)
    src_all = jnp.concatenate([edge_index_user_to_item[0],
                               edge_index_item_to_user[0] + N_NODE]
                              ).reshape(2 * ROWS_PER_REL, CHUNK)
    dst_all = jnp.concatenate([edge_index_user_to_item[1],
                               edge_index_item_to_user[1]]
                              ).reshape(2 * ROWS_PER_REL, CHUNK)
    zeros_rows = jnp.zeros((ROWS_PER_TILE, D), jnp.float32)

    agg_flat = _sc_agg(x_all, src_all, dst_all, zeros_rows)
    agg_item = agg_flat[0:N_NODE]
    agg_user = agg_flat[N_PAD:N_PAD + N_NODE]

    out_user, out_item = _dense(
        agg_user, x_user, agg_item, x_item,
        W_l_i2u.T, b_l_i2u.reshape(1, D), W_r_i2u.T,
        W_l_u2i.T, b_l_u2i.reshape(1, D), W_r_u2i.T)
    return out_user, out_item
